# baseline (speedup 1.0000x reference)
import functools

import jax
import jax.numpy as jnp
from jax import lax
from jax.experimental import pallas as pl
from jax.experimental.pallas import tpu as pltpu
from jax.experimental.pallas import tpu_sc as plsc

NC = 2
NS = 16
NW = NC * NS


def _make_deg_kernel(N, NCH, C):
    rpt = N // NS
    mesh = plsc.VectorSubcoreMesh(core_axis_name="c", subcore_axis_name="s")

    @functools.partial(
        pl.kernel,
        out_type=jax.ShapeDtypeStruct((NC, N), jnp.float32),
        mesh=mesh,
        scratch_types=(
            [pltpu.VMEM((NCH, C), jnp.int32)]
            + [pltpu.VMEM((C,), jnp.int32) for _ in range(4)]
            + [pltpu.VMEM((C,), jnp.float32),
               pltpu.VMEM((rpt,), jnp.float32),
               pltpu.VMEM_SHARED((N,), jnp.float32)]
            + [pltpu.SemaphoreType.DMA for _ in range(4)]
        ),
    )
    def deg_kernel(dst_hbm, out_hbm, dsts_v, db0, db1, db2, db3,
                   ones_v, zeros_v, acc_sh, sd0, sd1, sd2, sd3):
        dbuf = (db0, db1, db2, db3)
        semd = (sd0, sd1, sd2, sd3)
        cid = lax.axis_index("c")
        sid = lax.axis_index("s")
        wid = cid * NS + sid
        row0 = sid * rpt

        pltpu.sync_copy(dst_hbm.at[wid], dsts_v)

        for i in range(C // 16):
            ones_v[pl.ds(i * 16, 16)] = jnp.ones((16,), jnp.float32)

        def fill_zeros(i, _):
            zeros_v[pl.ds(i * 16, 16)] = jnp.zeros((16,), jnp.float32)
            return 0

        lax.fori_loop(0, rpt // 16, fill_zeros, 0)
        pltpu.sync_copy(zeros_v, acc_sh.at[pl.ds(row0, rpt)])
        plsc.subcore_barrier()

        def quad(j, _):
            for q in range(4):
                c = 4 * j + q

                @pl.when(c >= 4)
                def _():
                    pltpu.make_async_copy(ones_v, acc_sh.at[dbuf[q]],
                                          semd[q]).wait()

                for k in range(C // 16):
                    dbuf[q][pl.ds(16 * k, 16)] = dsts_v[c, pl.ds(16 * k, 16)]
                pltpu.async_copy(ones_v, acc_sh.at[dbuf[q]], semd[q],
                                 add=True)
            return 0

        lax.fori_loop(0, NCH // 4, quad, 0)
        for q in range(4):
            pltpu.make_async_copy(ones_v, acc_sh.at[dbuf[q]], semd[q]).wait()
        plsc.subcore_barrier()
        pltpu.sync_copy(acc_sh.at[pl.ds(row0, rpt)],
                        out_hbm.at[cid, pl.ds(row0, rpt)])

    return deg_kernel


def _make_prop_kernel(N, D, NCH, C):
    rpt = N // NS
    zrows = 16
    mesh = plsc.VectorSubcoreMesh(core_axis_name="c", subcore_axis_name="s")
    assert NCH % 4 == 0 and NCH >= 8

    @functools.partial(
        pl.kernel,
        out_type=jax.ShapeDtypeStruct((NC, N, D), jnp.float32),
        mesh=mesh,
        scratch_types=(
            [pltpu.VMEM((C,), jnp.int32) for _ in range(4)]
            + [pltpu.VMEM((C,), jnp.int32) for _ in range(4)]
            + [pltpu.VMEM((C, D), jnp.float32) for _ in range(2)]
            + [pltpu.VMEM((zrows, D), jnp.float32),
               pltpu.VMEM_SHARED((N, D), jnp.float32)]
            + [pltpu.SemaphoreType.DMA for _ in range(9)]
        ),
    )
    def prop_kernel(h_hbm, src_hbm, dst_hbm, out_hbm, *refs):
        isrc = refs[0:4]
        idst = refs[4:8]
        rows = refs[8:10]
        zeros_v = refs[10]
        acc_sh = refs[11]
        semi = refs[12:16]
        semg = refs[16:18]
        sems = refs[18:20]
        semz = refs[20]
        cid = lax.axis_index("c")
        sid = lax.axis_index("s")
        base = (cid * NS + sid) * (NCH * C)
        row0 = sid * rpt

        def start_idx(c, slot):
            pltpu.async_copy(src_hbm.at[pl.ds(base + c * C, C)],
                             isrc[slot], semi[slot])
            pltpu.async_copy(dst_hbm.at[pl.ds(base + c * C, C)],
                             idst[slot], semi[slot])

        def wait_idx(c, slot):
            pltpu.make_async_copy(src_hbm.at[pl.ds(base + c * C, C)],
                                  isrc[slot], semi[slot]).wait()
            pltpu.make_async_copy(dst_hbm.at[pl.ds(base + c * C, C)],
                                  idst[slot], semi[slot]).wait()

        def scatter_wait(slot):
            pltpu.make_async_copy(rows[slot % 2], acc_sh.at[idst[slot % 4]],
                                  sems[slot % 2]).wait()

        for q in range(3):
            start_idx(q, q)

        def fill_zeros(i, _):
            zeros_v[i, :] = jnp.zeros((D,), jnp.float32)
            return 0

        lax.fori_loop(0, zrows, fill_zeros, 0)

        def zcopy(i, _):
            pltpu.async_copy(zeros_v, acc_sh.at[pl.ds(row0 + i * zrows, zrows)],
                             semz)
            return 0

        nz = rpt // zrows
        lax.fori_loop(0, nz, zcopy, 0)

        wait_idx(0, 0)
        pltpu.async_copy(h_hbm.at[isrc[0]], rows[0], semg[0])

        def zdrain(i, _):
            pltpu.make_async_copy(
                zeros_v, acc_sh.at[pl.ds(row0, zrows)], semz).wait()
            return 0

        lax.fori_loop(0, nz, zdrain, 0)
        plsc.subcore_barrier()

        def quad(j, _):
            for q in range(4):
                c = 4 * j + q
                p = q % 2
                pltpu.make_async_copy(h_hbm.at[isrc[q]], rows[p],
                                      semg[p]).wait()
                pltpu.async_copy(rows[p], acc_sh.at[idst[q]], sems[p],
                                 add=True)

                @pl.when(c >= 1)
                def _():
                    scatter_wait(q - 1)

                @pl.when(c + 3 < NCH)
                def _():
                    start_idx(c + 3, (q + 3) % 4)

                @pl.when(c + 1 < NCH)
                def _():
                    iq = (q + 1) % 4
                    wait_idx(c + 1, iq)
                    pltpu.async_copy(h_hbm.at[isrc[iq]], rows[1 - p],
                                     semg[1 - p])

            return 0

        lax.fori_loop(0, NCH // 4, quad, 0)
        scatter_wait(NCH - 1)
        plsc.subcore_barrier()
        pltpu.sync_copy(acc_sh.at[pl.ds(row0, rpt)],
                        out_hbm.at[cid, pl.ds(row0, rpt)])

    return prop_kernel


def _norm_scale(degp, x, R=1024):
    N, D = x.shape

    def body(degp_ref, x_ref, h_ref, norm_ref):
        deg = degp_ref[0] + degp_ref[1]
        norm = jnp.where(deg > 0, lax.rsqrt(jnp.maximum(deg, 1.0)), 0.0)
        h_ref[...] = x_ref[...] * norm
        norm_ref[...] = norm

    return pl.pallas_call(
        body,
        grid=(N // R,),
        in_specs=[
            pl.BlockSpec((NC, R, 1), lambda i: (0, i, 0)),
            pl.BlockSpec((R, D), lambda i: (i, 0)),
        ],
        out_specs=[
            pl.BlockSpec((R, D), lambda i: (i, 0)),
            pl.BlockSpec((R, 1), lambda i: (i, 0)),
        ],
        out_shape=[
            jax.ShapeDtypeStruct((N, D), jnp.float32),
            jax.ShapeDtypeStruct((N, 1), jnp.float32),
        ],
    )(degp, x)


def _combine_scale2(p, norm, R=1024):
    _, N, D = p.shape

    def body(p_ref, norm_ref, o_ref):
        n = norm_ref[...]
        o_ref[...] = (p_ref[0] + p_ref[1]) * (n * n)

    return pl.pallas_call(
        body,
        grid=(N // R,),
        in_specs=[
            pl.BlockSpec((NC, R, D), lambda i: (0, i, 0)),
            pl.BlockSpec((R, 1), lambda i: (i, 0)),
        ],
        out_specs=pl.BlockSpec((R, D), lambda i: (i, 0)),
        out_shape=jax.ShapeDtypeStruct((N, D), jnp.float32),
    )(p, norm)


def _combine_linear(p, norm, W, b, R=1024):
    _, N, D = p.shape
    DO = W.shape[1]

    def body(p_ref, norm_ref, w_ref, b_ref, o_ref):
        h = (p_ref[0] + p_ref[1]) * norm_ref[...]
        o_ref[...] = (
            jnp.dot(h, w_ref[...], preferred_element_type=jnp.float32)
            + b_ref[...]
        )

    return pl.pallas_call(
        body,
        grid=(N // R,),
        in_specs=[
            pl.BlockSpec((NC, R, D), lambda i: (0, i, 0)),
            pl.BlockSpec((R, 1), lambda i: (i, 0)),
            pl.BlockSpec((D, DO), lambda i: (0, 0)),
            pl.BlockSpec((1, DO), lambda i: (0, 0)),
        ],
        out_specs=pl.BlockSpec((R, DO), lambda i: (i, 0)),
        out_shape=jax.ShapeDtypeStruct((N, DO), jnp.float32),
    )(p, norm, W, b.reshape(1, DO))


def kernel(x, edge_index, W, b):
    N, D = x.shape
    E = edge_index.shape[1]
    Cd = 128
    Cp = 128
    NP = ((N + NS * 128 - 1) // (NS * 128)) * (NS * 128)
    PT = -(-E // (NW * 4 * Cp)) * 4 * Cp
    if NP == N and PT * NW != E:
        NP += NS * 128
    EP = PT * NW

    src = edge_index[0]
    dst = edge_index[1]
    npad = EP - E
    if npad:
        pad_src = (jnp.arange(npad, dtype=jnp.int32) % N)
        pad_dst = N + (jnp.arange(npad, dtype=jnp.int32) % (NP - N))
        src = jnp.concatenate([src, pad_src])
        dst = jnp.concatenate([dst, pad_dst])
    dst3 = dst.reshape(NW, PT // Cd, Cd)
    xp = jnp.pad(x, ((0, NP - N), (0, 0))) if NP != N else x

    degp = _make_deg_kernel(NP, PT // Cd, Cd)(dst3).reshape(NC, NP, 1)
    h1, norm = _norm_scale(degp, xp)
    prop = _make_prop_kernel(NP, D, PT // Cp, Cp)
    p1 = prop(h1, src, dst)
    h2 = _combine_scale2(p1, norm)
    p2 = prop(h2, src, dst)
    out = _combine_linear(p2, norm, W, b)
    return out[:N] if NP != N else out


# speedup vs baseline: 1.1607x; 1.1607x over previous
import functools

import jax
import jax.numpy as jnp
from jax import lax
from jax.experimental import pallas as pl
from jax.experimental.pallas import tpu as pltpu
from jax.experimental.pallas import tpu_sc as plsc

NC = 2
NS = 16
NW = NC * NS


def _make_deg_kernel(N, NCH, C):
    rpt = N // NS
    mesh = plsc.VectorSubcoreMesh(core_axis_name="c", subcore_axis_name="s")

    @functools.partial(
        pl.kernel,
        out_type=jax.ShapeDtypeStruct((NC, N), jnp.float32),
        mesh=mesh,
        scratch_types=(
            [pltpu.VMEM((NCH, C), jnp.int32)]
            + [pltpu.VMEM((C,), jnp.int32) for _ in range(4)]
            + [pltpu.VMEM((C,), jnp.float32),
               pltpu.VMEM((rpt,), jnp.float32),
               pltpu.VMEM_SHARED((N,), jnp.float32)]
            + [pltpu.SemaphoreType.DMA for _ in range(4)]
        ),
    )
    def deg_kernel(dst_hbm, out_hbm, dsts_v, db0, db1, db2, db3,
                   ones_v, zeros_v, acc_sh, sd0, sd1, sd2, sd3):
        dbuf = (db0, db1, db2, db3)
        semd = (sd0, sd1, sd2, sd3)
        cid = lax.axis_index("c")
        sid = lax.axis_index("s")
        wid = cid * NS + sid
        row0 = sid * rpt

        pltpu.sync_copy(dst_hbm.at[wid], dsts_v)

        for i in range(C // 16):
            ones_v[pl.ds(i * 16, 16)] = jnp.ones((16,), jnp.float32)

        def fill_zeros(i, _):
            zeros_v[pl.ds(i * 16, 16)] = jnp.zeros((16,), jnp.float32)
            return 0

        lax.fori_loop(0, rpt // 16, fill_zeros, 0)
        pltpu.sync_copy(zeros_v, acc_sh.at[pl.ds(row0, rpt)])
        plsc.subcore_barrier()

        def quad(j, _):
            for q in range(4):
                c = 4 * j + q

                @pl.when(c >= 4)
                def _():
                    pltpu.make_async_copy(ones_v, acc_sh.at[dbuf[q]],
                                          semd[q]).wait()

                for k in range(C // 16):
                    dbuf[q][pl.ds(16 * k, 16)] = dsts_v[c, pl.ds(16 * k, 16)]
                pltpu.async_copy(ones_v, acc_sh.at[dbuf[q]], semd[q],
                                 add=True)
            return 0

        lax.fori_loop(0, NCH // 4, quad, 0)
        for q in range(4):
            pltpu.make_async_copy(ones_v, acc_sh.at[dbuf[q]], semd[q]).wait()
        plsc.subcore_barrier()
        pltpu.sync_copy(acc_sh.at[pl.ds(row0, rpt)],
                        out_hbm.at[cid, pl.ds(row0, rpt)])

    return deg_kernel


def _make_prop_kernel(N, D, NCH, C):
    rpt = N // NS
    zrows = 16
    mesh = plsc.VectorSubcoreMesh(core_axis_name="c", subcore_axis_name="s")
    assert NCH % 4 == 0 and NCH >= 8

    @functools.partial(
        pl.kernel,
        out_type=jax.ShapeDtypeStruct((NC, N, D), jnp.float32),
        mesh=mesh,
        scratch_types=(
            [pltpu.VMEM((C,), jnp.int32) for _ in range(4)]
            + [pltpu.VMEM((C,), jnp.int32) for _ in range(4)]
            + [pltpu.VMEM((C, D), jnp.float32) for _ in range(2)]
            + [pltpu.VMEM((zrows, D), jnp.float32),
               pltpu.VMEM_SHARED((N, D), jnp.float32)]
            + [pltpu.SemaphoreType.DMA for _ in range(9)]
        ),
    )
    def prop_kernel(h_hbm, src_hbm, dst_hbm, out_hbm, *refs):
        isrc = refs[0:4]
        idst = refs[4:8]
        rows = refs[8:10]
        zeros_v = refs[10]
        acc_sh = refs[11]
        semi = refs[12:16]
        semg = refs[16:18]
        sems = refs[18:20]
        semz = refs[20]
        cid = lax.axis_index("c")
        sid = lax.axis_index("s")
        base = (cid * NS + sid) * (NCH * C)
        row0 = sid * rpt

        def start_idx(c, slot):
            pltpu.async_copy(src_hbm.at[pl.ds(base + c * C, C)],
                             isrc[slot], semi[slot])
            pltpu.async_copy(dst_hbm.at[pl.ds(base + c * C, C)],
                             idst[slot], semi[slot])

        def wait_idx(c, slot):
            pltpu.make_async_copy(src_hbm.at[pl.ds(base + c * C, C)],
                                  isrc[slot], semi[slot]).wait()
            pltpu.make_async_copy(dst_hbm.at[pl.ds(base + c * C, C)],
                                  idst[slot], semi[slot]).wait()

        for q in range(4):
            start_idx(q, q)

        def fill_zeros(i, _):
            zeros_v[i, :] = jnp.zeros((D,), jnp.float32)
            return 0

        lax.fori_loop(0, zrows, fill_zeros, 0)

        def zcopy(i, _):
            pltpu.async_copy(zeros_v, acc_sh.at[pl.ds(row0 + i * zrows, zrows)],
                             semz)
            return 0

        nz = rpt // zrows
        lax.fori_loop(0, nz, zcopy, 0)

        for q in range(2):
            wait_idx(q, q)
            pltpu.async_copy(h_hbm.at[isrc[q]], rows[q], semg[q])

        def zdrain(i, _):
            pltpu.make_async_copy(
                zeros_v, acc_sh.at[pl.ds(row0, zrows)], semz).wait()
            return 0

        lax.fori_loop(0, nz, zdrain, 0)
        plsc.subcore_barrier()

        def quad(j, _):
            for q in range(4):
                c = 4 * j + q
                p = q % 2
                pltpu.make_async_copy(h_hbm.at[isrc[q]], rows[p],
                                      semg[p]).wait()
                pltpu.sync_copy(rows[p], acc_sh.at[idst[q]], add=True)

                @pl.when(c + 4 < NCH)
                def _():
                    start_idx(c + 4, q)

                @pl.when(c + 2 < NCH)
                def _():
                    iq = (q + 2) % 4
                    wait_idx(c + 2, iq)
                    pltpu.async_copy(h_hbm.at[isrc[iq]], rows[p], semg[p])

            return 0

        lax.fori_loop(0, NCH // 4, quad, 0)
        plsc.subcore_barrier()
        pltpu.sync_copy(acc_sh.at[pl.ds(row0, rpt)],
                        out_hbm.at[cid, pl.ds(row0, rpt)])

    return prop_kernel


def _norm_scale(degp, x, R=1024):
    N, D = x.shape

    def body(degp_ref, x_ref, h_ref, norm_ref):
        deg = degp_ref[0] + degp_ref[1]
        norm = jnp.where(deg > 0, lax.rsqrt(jnp.maximum(deg, 1.0)), 0.0)
        h_ref[...] = x_ref[...] * norm
        norm_ref[...] = norm

    return pl.pallas_call(
        body,
        grid=(N // R,),
        in_specs=[
            pl.BlockSpec((NC, R, 1), lambda i: (0, i, 0)),
            pl.BlockSpec((R, D), lambda i: (i, 0)),
        ],
        out_specs=[
            pl.BlockSpec((R, D), lambda i: (i, 0)),
            pl.BlockSpec((R, 1), lambda i: (i, 0)),
        ],
        out_shape=[
            jax.ShapeDtypeStruct((N, D), jnp.float32),
            jax.ShapeDtypeStruct((N, 1), jnp.float32),
        ],
    )(degp, x)


def _combine_scale2(p, norm, R=1024):
    _, N, D = p.shape

    def body(p_ref, norm_ref, o_ref):
        n = norm_ref[...]
        o_ref[...] = (p_ref[0] + p_ref[1]) * (n * n)

    return pl.pallas_call(
        body,
        grid=(N // R,),
        in_specs=[
            pl.BlockSpec((NC, R, D), lambda i: (0, i, 0)),
            pl.BlockSpec((R, 1), lambda i: (i, 0)),
        ],
        out_specs=pl.BlockSpec((R, D), lambda i: (i, 0)),
        out_shape=jax.ShapeDtypeStruct((N, D), jnp.float32),
    )(p, norm)


def _combine_linear(p, norm, W, b, R=1024):
    _, N, D = p.shape
    DO = W.shape[1]

    def body(p_ref, norm_ref, w_ref, b_ref, o_ref):
        h = (p_ref[0] + p_ref[1]) * norm_ref[...]
        o_ref[...] = (
            jnp.dot(h, w_ref[...], preferred_element_type=jnp.float32)
            + b_ref[...]
        )

    return pl.pallas_call(
        body,
        grid=(N // R,),
        in_specs=[
            pl.BlockSpec((NC, R, D), lambda i: (0, i, 0)),
            pl.BlockSpec((R, 1), lambda i: (i, 0)),
            pl.BlockSpec((D, DO), lambda i: (0, 0)),
            pl.BlockSpec((1, DO), lambda i: (0, 0)),
        ],
        out_specs=pl.BlockSpec((R, DO), lambda i: (i, 0)),
        out_shape=jax.ShapeDtypeStruct((N, DO), jnp.float32),
    )(p, norm, W, b.reshape(1, DO))


def kernel(x, edge_index, W, b):
    N, D = x.shape
    E = edge_index.shape[1]
    Cd = 128
    Cp = 128
    NP = ((N + NS * 128 - 1) // (NS * 128)) * (NS * 128)
    PT = -(-E // (NW * 4 * Cp)) * 4 * Cp
    if NP == N and PT * NW != E:
        NP += NS * 128
    EP = PT * NW

    src = edge_index[0]
    dst = edge_index[1]
    npad = EP - E
    if npad:
        pad_src = (jnp.arange(npad, dtype=jnp.int32) % N)
        pad_dst = N + (jnp.arange(npad, dtype=jnp.int32) % (NP - N))
        src = jnp.concatenate([src, pad_src])
        dst = jnp.concatenate([dst, pad_dst])
    dst3 = dst.reshape(NW, PT // Cd, Cd)
    xp = jnp.pad(x, ((0, NP - N), (0, 0))) if NP != N else x

    degp = _make_deg_kernel(NP, PT // Cd, Cd)(dst3).reshape(NC, NP, 1)
    h1, norm = _norm_scale(degp, xp)
    prop = _make_prop_kernel(NP, D, PT // Cp, Cp)
    p1 = prop(h1, src, dst)
    h2 = _combine_scale2(p1, norm)
    p2 = prop(h2, src, dst)
    out = _combine_linear(p2, norm, W, b)
    return out[:N] if NP != N else out
